# baseline (speedup 1.0000x reference)
import math

import jax
import jax.numpy as jnp
from jax.experimental import pallas as pl
from jax.experimental.pallas import tpu as pltpu

N = 768
_REQ = int((1 - 0.1) * N * N)
_K = math.ceil(_REQ / N)
_BT = 512


def _fc_kernel(a_ref, v_ref, x_ref, out_ref, wt_ref, xb_ref):
    @pl.when(pl.program_id(0) == 0)
    def _build_wt():
        a = a_ref[...]
        e = jnp.exp(a - jnp.max(a))
        atk = jnp.clip((_K / jnp.sum(e)) * e, 0.0, 1.0)
        w = jnp.transpose((v_ref[...] * atk).astype(jnp.bfloat16))
        row = jax.lax.broadcasted_iota(jnp.int32, (N, 1), 0)
        for b in range(10):
            amt = 1 << b
            rolled = jnp.concatenate([w[:, N - amt:], w[:, :N - amt]], axis=1)
            w = jnp.where((row & amt) != 0, rolled, w)
        wt_ref[...] = w

    xb_ref[...] = x_ref[...].astype(jnp.bfloat16)
    out_ref[...] = jnp.dot(xb_ref[...], wt_ref[...],
                           preferred_element_type=jnp.float32)


@jax.jit
def kernel(x, V, alpha):
    batch = x.shape[0]
    return pl.pallas_call(
        _fc_kernel,
        grid=(batch // _BT,),
        in_specs=[
            pl.BlockSpec((N, 1), lambda i: (0, 0)),
            pl.BlockSpec((N, N), lambda i: (0, 0)),
            pl.BlockSpec((_BT, N), lambda i: (i, 0)),
        ],
        out_specs=pl.BlockSpec((_BT, N), lambda i: (i, 0)),
        out_shape=jax.ShapeDtypeStruct((batch, N), jnp.float32),
        scratch_shapes=[pltpu.VMEM((N, N), jnp.bfloat16),
                        pltpu.VMEM((_BT, N), jnp.bfloat16)],
    )(alpha.reshape(N, 1), V, x)


# speedup vs baseline: 1.1125x; 1.1125x over previous
import math

import jax
import jax.numpy as jnp
from jax.experimental import pallas as pl
from jax.experimental.pallas import tpu as pltpu

N = 768
_REQ = int((1 - 0.1) * N * N)
_K = math.ceil(_REQ / N)
_BT = 1024


def _fc_kernel(a_ref, v_ref, x_ref, out_ref, wt_ref):
    @pl.when(pl.program_id(0) == 0)
    def _build_wt():
        a = a_ref[...]
        e = jnp.exp(a - jnp.max(a))
        atk = jnp.clip((_K / jnp.sum(e)) * e, 0.0, 1.0)
        w = jnp.transpose((v_ref[...] * atk).astype(jnp.bfloat16))
        row = jax.lax.broadcasted_iota(jnp.int32, (N, 1), 0)
        for b in range(10):
            amt = 1 << b
            rolled = jnp.concatenate([w[:, N - amt:], w[:, :N - amt]], axis=1)
            w = jnp.where((row & amt) != 0, rolled, w)
        wt_ref[...] = w

    out_ref[...] = jnp.dot(x_ref[...].astype(jnp.bfloat16), wt_ref[...],
                           preferred_element_type=jnp.float32)


@jax.jit
def kernel(x, V, alpha):
    batch = x.shape[0]
    return pl.pallas_call(
        _fc_kernel,
        grid=(batch // _BT,),
        in_specs=[
            pl.BlockSpec((N, 1), lambda i: (0, 0)),
            pl.BlockSpec((N, N), lambda i: (0, 0)),
            pl.BlockSpec((_BT, N), lambda i: (i, 0)),
        ],
        out_specs=pl.BlockSpec((_BT, N), lambda i: (i, 0)),
        out_shape=jax.ShapeDtypeStruct((batch, N), jnp.float32),
        scratch_shapes=[pltpu.VMEM((N, N), jnp.bfloat16)],
    )(alpha.reshape(N, 1), V, x)


# speedup vs baseline: 1.1906x; 1.0702x over previous
import math

import jax
import jax.numpy as jnp
from jax.experimental import pallas as pl
from jax.experimental.pallas import tpu as pltpu

N = 768
_REQ = int((1 - 0.1) * N * N)
_K = math.ceil(_REQ / N)
_BT = 1024


def _fc_kernel(a_ref, v_ref, x_ref, out_ref, wt_ref):
    @pl.when(pl.program_id(0) == 0)
    def _build_wt():
        a = a_ref[...]
        e = jnp.exp(a - jnp.max(a))
        atk = jnp.clip((_K / jnp.sum(e)) * e, 0.0, 1.0)
        w = (v_ref[...] * atk).astype(jnp.bfloat16)
        col = jax.lax.broadcasted_iota(jnp.int32, (1, N), 1)
        for b in range(10):
            amt = 1 << b
            rolled = jnp.concatenate([w[N - amt:, :], w[:N - amt, :]], axis=0)
            w = jnp.where((col & amt) != 0, rolled, w)
        wt_ref[...] = w

    out_ref[...] = jax.lax.dot_general(
        x_ref[...].astype(jnp.bfloat16), wt_ref[...],
        (((1,), (1,)), ((), ())), preferred_element_type=jnp.float32)


@jax.jit
def kernel(x, V, alpha):
    batch = x.shape[0]
    return pl.pallas_call(
        _fc_kernel,
        grid=(batch // _BT,),
        in_specs=[
            pl.BlockSpec((N, 1), lambda i: (0, 0)),
            pl.BlockSpec((N, N), lambda i: (0, 0)),
            pl.BlockSpec((_BT, N), lambda i: (i, 0)),
        ],
        out_specs=pl.BlockSpec((_BT, N), lambda i: (i, 0)),
        out_shape=jax.ShapeDtypeStruct((batch, N), jnp.float32),
        scratch_shapes=[pltpu.VMEM((N, N), jnp.bfloat16)],
    )(alpha.reshape(N, 1), V, x)
